# baseline (speedup 1.0000x reference)
import functools

import jax
import jax.numpy as jnp
from jax import lax
from jax.experimental import pallas as pl
from jax.experimental.pallas import tpu as pltpu
from jax.experimental.pallas import tpu_sc as plsc

_DIM = 64
_BATCH = 16384
_HIST = 200
_B = _BATCH * _HIST
_NW = 32
_BPW = _B // _NW
_C = 512
_STEPS = _BPW // _C

_mesh = plsc.VectorSubcoreMesh(core_axis_name="c", subcore_axis_name="s")


@functools.partial(
    pl.kernel,
    mesh=_mesh,
    out_type=jax.ShapeDtypeStruct((_B, _DIM), jnp.float32),
    scratch_types=[
        pltpu.VMEM((_C,), jnp.int32),
        pltpu.VMEM((_C, _DIM), jnp.float32),
        pltpu.SemaphoreType.DMA,
    ],
    compiler_params=pltpu.CompilerParams(use_tc_tiling_on_sc=False),
)
def _emb(x_hbm, tab_hbm, out_hbm, idx_v, rows_v, gsem):
    wid = lax.axis_index("s") * 2 + lax.axis_index("c")
    base = wid * _BPW

    def body(g, carry):
        off = base + g * _C
        pltpu.sync_copy(x_hbm.at[pl.ds(off, _C)], idx_v)
        pltpu.async_copy(tab_hbm.at[idx_v], rows_v, gsem).wait()
        pltpu.sync_copy(rows_v, out_hbm.at[pl.ds(off, _C)])
        return carry

    lax.fori_loop(0, _STEPS, body, 0)


def kernel(x, table):
    flat = x.reshape(_B)
    out = _emb(flat, table)
    return out.reshape(_BATCH, _HIST, _DIM)


# speedup vs baseline: 1.0559x; 1.0559x over previous
import functools

import jax
import jax.numpy as jnp
from jax import lax
from jax.experimental import pallas as pl
from jax.experimental.pallas import tpu as pltpu
from jax.experimental.pallas import tpu_sc as plsc

_DIM = 64
_BATCH = 16384
_HIST = 200
_B = _BATCH * _HIST
_NW = 32
_BPW = _B // _NW
_C = 512
_STEPS = _BPW // _C

_mesh = plsc.VectorSubcoreMesh(core_axis_name="c", subcore_axis_name="s")


@functools.partial(
    pl.kernel,
    mesh=_mesh,
    out_type=jax.ShapeDtypeStruct((_B, _DIM), jnp.float32),
    scratch_types=[
        pltpu.VMEM((2, _C), jnp.int32),
        pltpu.VMEM((2, _C, _DIM), jnp.float32),
        pltpu.SemaphoreType.DMA,
        pltpu.SemaphoreType.DMA,
    ],
    compiler_params=pltpu.CompilerParams(use_tc_tiling_on_sc=False),
)
def _emb(x_hbm, tab_hbm, out_hbm, idx_v, rows_v, gsem, osem):
    wid = lax.axis_index("s") * 2 + lax.axis_index("c")
    base = wid * _BPW

    def idx_load(g, slot):
        pltpu.sync_copy(x_hbm.at[pl.ds(base + g * _C, _C)], idx_v.at[slot])

    def gather_start(slot):
        return pltpu.async_copy(tab_hbm.at[idx_v.at[slot]], rows_v.at[slot],
                                gsem)

    def store_start(g, slot):
        return pltpu.async_copy(rows_v.at[slot],
                                out_hbm.at[pl.ds(base + g * _C, _C)], osem)

    def wait(sem, slot):
        pltpu.make_async_copy(tab_hbm.at[pl.ds(0, _C)], rows_v.at[slot],
                              sem).wait()

    idx_load(0, 0)
    gather_start(0)
    idx_load(1, 1)
    gather_start(1)
    wait(gsem, 0)
    store_start(0, 0)

    def body(go, carry):
        for b in range(2):
            g = go * 2 + b
            wait(osem, b)
            idx_load(g, b)
            gather_start(b)
            wait(gsem, 1 - b)
            store_start(g - 1, 1 - b)
        return carry

    lax.fori_loop(1, _STEPS // 2, body, 0)

    wait(gsem, 1)
    store_start(_STEPS - 1, 1)
    wait(osem, 0)
    wait(osem, 1)


def kernel(x, table):
    flat = x.reshape(_B)
    out = _emb(flat, table)
    return out.reshape(_BATCH, _HIST, _DIM)


# speedup vs baseline: 1.0576x; 1.0016x over previous
import functools

import jax
import jax.numpy as jnp
from jax import lax
from jax.experimental import pallas as pl
from jax.experimental.pallas import tpu as pltpu
from jax.experimental.pallas import tpu_sc as plsc

_VOCAB = 1000000
_DIM = 64
_BATCH = 16384
_HIST = 200
_NW = 32
_RPW = _BATCH // _NW
_NR = 4
_STEPS = _RPW // _NR

_mesh = plsc.VectorSubcoreMesh(core_axis_name="c", subcore_axis_name="s")


@functools.partial(
    pl.kernel,
    mesh=_mesh,
    out_type=jax.ShapeDtypeStruct((_BATCH, _HIST, _DIM), jnp.float32),
    scratch_types=[
        pltpu.VMEM((2, _NR, _HIST), jnp.int32),
        pltpu.VMEM((2, _NR, _HIST, _DIM), jnp.float32),
        pltpu.SemaphoreType.DMA,
        pltpu.SemaphoreType.DMA,
    ],
    compiler_params=pltpu.CompilerParams(use_tc_tiling_on_sc=False),
)
def _emb(x_hbm, tab_hbm, out_hbm, idx_v, rows_v, gsem, osem):
    wid = lax.axis_index("s") * 2 + lax.axis_index("c")
    base = wid * _RPW

    def idx_load(g, slot):
        pltpu.sync_copy(x_hbm.at[pl.ds(base + g * _NR, _NR)], idx_v.at[slot])

    def gather_start(slot):
        for r in range(_NR):
            pltpu.async_copy(tab_hbm.at[idx_v.at[slot, r]],
                             rows_v.at[slot, r], gsem)

    def store_start(g, slot):
        pltpu.async_copy(rows_v.at[slot],
                         out_hbm.at[pl.ds(base + g * _NR, _NR)], osem)

    def wait(sem, slot):
        pltpu.make_async_copy(out_hbm.at[pl.ds(0, _NR)], rows_v.at[slot],
                              sem).wait()

    idx_load(0, 0)
    gather_start(0)
    idx_load(1, 1)
    gather_start(1)
    wait(gsem, 0)
    store_start(0, 0)

    def body(go, carry):
        for b in range(2):
            g = go * 2 + b
            wait(osem, b)
            idx_load(g, b)
            gather_start(b)
            wait(gsem, 1 - b)
            store_start(g - 1, 1 - b)
        return carry

    lax.fori_loop(1, _STEPS // 2, body, 0)

    wait(gsem, 1)
    store_start(_STEPS - 1, 1)
    wait(osem, 0)
    wait(osem, 1)


def kernel(x, table):
    return _emb(x, table)
